# Initial kernel scaffold; baseline (speedup 1.0000x reference)
import jax
import jax.numpy as jnp
from jax.experimental import pallas as pl


def kernel(x, pos, batch, params):
    raise NotImplementedError("write your pallas kernel here")


# speedup vs baseline: 9.3297x; 9.3297x over previous
import functools

import jax
import jax.numpy as jnp
from jax.experimental import pallas as pl

_B = 10
_N = 1000
_K = 16
_D = 128
_CH = 64
_NCLS = 5
_NEG = -1.0e30

_dot = functools.partial(jnp.dot, preferred_element_type=jnp.float32)


def _relu(v):
    return jnp.maximum(v, 0.0)


def _edge_conv(pts, fts, u_row, Mu, Md, Mj, d1, M2, d2b, M3, d3b, Ms, dsb):
    sq = jnp.sum(pts * pts, axis=1, keepdims=True)
    g = jax.lax.dot_general(pts, pts, (((1,), (1,)), ((), ())),
                            preferred_element_type=jnp.float32)
    row_i = jax.lax.broadcasted_iota(jnp.int32, (_N, _N), 0)
    col_i = jax.lax.broadcasted_iota(jnp.int32, (_N, _N), 1)
    eye = row_i == col_i
    sq_row = jnp.sum(jnp.where(eye, g, 0.0), axis=0, keepdims=True)
    neg = 2.0 * g - (sq + sq_row)
    neg = jnp.where(eye, _NEG, neg)

    a = _dot(fts, Md) + (_dot(u_row, Mu) + d1)
    b_pre = _dot(fts, Mj)

    aggr = None
    for kk in range(_K):
        m = jnp.max(neg, axis=1, keepdims=True)
        idx = jnp.min(jnp.where(neg == m, col_i, _N), axis=1,
                      keepdims=True)
        sel = col_i == idx
        bj = _dot(sel.astype(jnp.float32), b_pre)
        h = _relu(a + bj)
        h = _relu(_dot(h, M2) + d2b)
        h = _relu(_dot(h, M3) + d3b)
        aggr = h if aggr is None else jnp.maximum(aggr, h)
        if kk < _K - 1:
            neg = jnp.where(sel, _NEG, neg)

    return _relu(aggr + _dot(fts, Ms) + dsb)


def _body(x_ref, pos_ref, s_in, b_in, M0, b0,
          Mu0, Md0, Mj0, d10, M20, d20, M30, d30, Ms0, ds0,
          GA0, GB0, gb0,
          Mu1, Md1, Mj1, d11, M21, d21, M31, d31, Ms1, ds1,
          GA1, GB1, gb1,
          Mfc, bfc, Mout, bout, out_ref):
    fts = x_ref[0] * s_in[...] + b_in[...]
    u = jnp.sum(fts, axis=0, keepdims=True) * (1.0 / _N)
    u = _relu(_dot(u, M0[...]) + b0[...])

    fts = _edge_conv(pos_ref[0], fts, u, Mu0[...], Md0[...], Mj0[...],
                     d10[...], M20[...], d20[...], M30[...], d30[...],
                     Ms0[...], ds0[...])
    pooled = jnp.sum(fts, axis=0, keepdims=True) * (1.0 / _N)
    u = _relu(_dot(u, GA0[...]) + _dot(pooled, GB0[...]) + gb0[...])

    fts = _edge_conv(fts, fts, u, Mu1[...], Md1[...], Mj1[...],
                     d11[...], M21[...], d21[...], M31[...], d31[...],
                     Ms1[...], ds1[...])
    pooled = jnp.sum(fts, axis=0, keepdims=True) * (1.0 / _N)
    u = _relu(_dot(u, GA1[...]) + _dot(pooled, GB1[...]) + gb1[...])

    h = _relu(_dot(u, Mfc[...]) + bfc[...])
    out_ref[0] = _dot(h, Mout[...]) + bout[...]


def kernel(x, pos, batch, params):
    del batch
    inv = (1.0 + 1e-5) ** -0.5

    def fold_w(W, g):
        return ((g * inv)[:, None] * W).T

    def row(v):
        return v[None, :]

    ins = []
    ins += [row(params['bn_in_g'] * inv), row(params['bn_in_b'])]
    c0 = params['g0_bng'] * inv
    ins += [fold_w(params['g0_W'], params['g0_bng']),
            row(c0 * params['g0_b'] + params['g0_bnb'])]
    for l in range(2):
        p = params['convs'][l]
        in_ch = _D if l == 0 else _CH
        W1 = p['W1']
        Wu, Wi, Wj = (W1[:, :in_ch], W1[:, in_ch:2 * in_ch],
                      W1[:, 2 * in_ch:])
        ins += [fold_w(Wu, p['bn1g']), fold_w(Wi - Wj, p['bn1g']),
                fold_w(Wj, p['bn1g']), row(p['bn1b']),
                fold_w(p['W2'], p['bn2g']), row(p['bn2b']),
                fold_w(p['W3'], p['bn3g']), row(p['bn3b']),
                fold_w(p['Ws'], p['bnsg']), row(p['bnsb'])]
        gp = params['glob'][l]
        prev = _D if l == 0 else _CH
        cg = gp['g'] * inv
        ins += [fold_w(gp['W'][:, :prev], gp['g']),
                fold_w(gp['W'][:, prev:], gp['g']),
                row(cg * gp['b'] + gp['b2'])]
    ins += [params['fc_W'].T, row(params['fc_b']),
            params['out_W'].T, row(params['out_b'])]

    x3 = x.reshape(_B, _N, _D)
    pos3 = pos.reshape(_B, _N, pos.shape[-1])

    in_specs = [
        pl.BlockSpec((1, _N, _D), lambda gi: (gi, 0, 0)),
        pl.BlockSpec((1, _N, pos.shape[-1]), lambda gi: (gi, 0, 0)),
    ] + [pl.BlockSpec(w.shape, lambda gi: (0,) * w.ndim) for w in ins]

    out = pl.pallas_call(
        _body,
        grid=(_B,),
        in_specs=in_specs,
        out_specs=pl.BlockSpec((1, 1, _NCLS), lambda gi: (gi, 0, 0)),
        out_shape=jax.ShapeDtypeStruct((_B, 1, _NCLS), jnp.float32),
    )(x3, pos3, *ins)
    return out.reshape(_B, _NCLS)
